# baseline (speedup 1.0000x reference)
import jax
import jax.numpy as jnp
from jax.experimental import pallas as pl
from jax.experimental.pallas import tpu as pltpu

_N = 20000
_NP = 32768
_R0 = 256
_K = 1000
_KP = 1024
_T = 0.5


def _flat(r):
    ri = jax.lax.broadcasted_iota(jnp.int32, (r, 128), 0)
    ci = jax.lax.broadcasted_iota(jnp.int32, (r, 128), 1)
    return ri * 128 + ci


def _exchange(arrs, lj, dirm):
    score, idx = arrs[0], arrs[1]
    r = score.shape[0]
    fi = _flat(r)
    d = jax.lax.shift_left(jnp.int32(1), jnp.int32(lj))
    bit = (fi & d) != 0

    def lane_partners(xs):
        return tuple(
            jnp.where(bit, pltpu.roll(a, d, 1), pltpu.roll(a, 128 - d, 1))
            for a in xs)

    def row_partners(xs):
        m = jax.lax.shift_right_logical(d, 7)
        return tuple(
            jnp.where(bit, pltpu.roll(a, m, 0), pltpu.roll(a, r - m, 0))
            for a in xs)

    partners = jax.lax.cond(jnp.int32(lj) < 7, lane_partners, row_partners,
                            tuple(arrs))
    ps, pi = partners[0], partners[1]
    pref = (score > ps) | ((score == ps) & (idx < pi))
    keep_mine = ~(pref ^ dirm ^ bit)
    return [jnp.where(keep_mine, a, p) for a, p in zip(arrs, partners)]


def _topk_sort(arrs):
    n = len(arrs)

    def outer(lk, t):
        ksz = jax.lax.shift_left(jnp.int32(1), lk)
        dirm = (_flat(_R0) & ksz) == 0

        def inner(s, t2):
            return tuple(_exchange(list(t2), lk - 1 - s, dirm))

        return jax.lax.fori_loop(0, lk, inner, t)

    arrs = list(jax.lax.fori_loop(1, 11, outer, tuple(arrs)))

    size = _NP
    while size > _KP:
        r = arrs[0].shape[0]
        arrs = _exchange(arrs, 10, jnp.full((r, 128), True))
        arrs = [a.reshape(r // 16, 2, 8, 128)[:, 0].reshape(r // 2, 128)
                for a in arrs]
        size //= 2
        r //= 2
        final = size == _KP
        dirm = (jnp.full((r, 128), True) if final
                else (_flat(r) & _KP) == 0)

        def merge(s, t2, dirm=dirm):
            return tuple(_exchange(list(t2), 9 - s, dirm))

        arrs = list(jax.lax.fori_loop(0, 10, merge, tuple(arrs)))
    return arrs


def _rowform(a):
    return jnp.concatenate([a[i:i + 1, :] for i in range(8)], axis=1)


def _colform(a):
    b = jnp.broadcast_to(a[:, None, :], (8, 128, 128)).reshape(_KP, 128)
    ri = jax.lax.broadcasted_iota(jnp.int32, (_KP, 128), 0)
    ci = jax.lax.broadcasted_iota(jnp.int32, (_KP, 128), 1)
    sel = ci == (ri & 127)
    return jnp.sum(jnp.where(sel, b, 0.0), axis=1, keepdims=True)


def _body(score_ref, x1_ref, y1_ref, x2_ref, y2_ref, init_ref, out_ref):
    idx0 = _flat(_R0)
    arrs = [score_ref[...], idx0,
            x1_ref[...], y1_ref[...], x2_ref[...], y2_ref[...]]
    s, _, x1, y1, x2, y2 = _topk_sort(arrs)

    sr = _rowform(s)
    x1r = _rowform(x1)
    y1r = _rowform(y1)
    x2r = _rowform(x2)
    y2r = _rowform(y2)
    x1c = _colform(x1)
    y1c = _colform(y1)
    x2c = _colform(x2)
    y2c = _colform(y2)

    area_c = (x2c - x1c) * (y2c - y1c)
    area_r = (x2r - x1r) * (y2r - y1r)
    xx1 = jnp.maximum(x1c, x1r)
    yy1 = jnp.maximum(y1c, y1r)
    xx2 = jnp.minimum(x2c, x2r)
    yy2 = jnp.minimum(y2c, y2r)
    iw = jnp.clip(xx2 - xx1, 0.0, None)
    ih = jnp.clip(yy2 - yy1, 0.0, None)
    inter = iw * ih
    union = area_c + area_r - inter
    iou = inter / (union + 1e-8)

    ii = jax.lax.broadcasted_iota(jnp.int32, (_KP, _KP), 0)
    jj = jax.lax.broadcasted_iota(jnp.int32, (_KP, _KP), 1)
    sup = jnp.where((iou > _T) & (ii < jj), 1.0, 0.0)

    init = init_ref[...]

    def cond(c):
        return c[1]

    def body(c):
        keep, _ = c
        hits = jax.lax.dot_general(
            keep, sup, (((1,), (0,)), ((), ())),
            preferred_element_type=jnp.float32)
        new = jnp.where(hits == 0.0, init, 0.0)
        return new, jnp.any(new != keep)

    keep, _ = jax.lax.while_loop(cond, body, (init, jnp.bool_(True)))

    out_ref[0:1, :] = x1r * keep
    out_ref[1:2, :] = y1r * keep
    out_ref[2:3, :] = x2r * keep
    out_ref[3:4, :] = y2r * keep
    out_ref[4:5, :] = sr * keep
    out_ref[5:8, :] = jnp.zeros((3, _KP), jnp.float32)


def kernel(boxes, scores, k):
    pad = _NP - _N
    sp = jnp.pad(scores, (0, pad), constant_values=-1.0).reshape(_R0, 128)
    x1 = jnp.pad(boxes[:, 0], (0, pad)).reshape(_R0, 128)
    y1 = jnp.pad(boxes[:, 1], (0, pad)).reshape(_R0, 128)
    x2 = jnp.pad(boxes[:, 2], (0, pad)).reshape(_R0, 128)
    y2 = jnp.pad(boxes[:, 3], (0, pad)).reshape(_R0, 128)
    init = (jnp.arange(_KP) < k).astype(jnp.float32).reshape(1, _KP)

    out = pl.pallas_call(
        _body,
        out_shape=jax.ShapeDtypeStruct((8, _KP), jnp.float32),
    )(sp, x1, y1, x2, y2, init)

    return out.T[:_K, :5]


# speedup vs baseline: 2.0681x; 2.0681x over previous
import jax
import jax.numpy as jnp
from jax.experimental import pallas as pl
from jax.experimental.pallas import tpu as pltpu

_N = 20000
_NP = 32768
_R0 = 256
_K = 1000
_KP = 1024
_T = 0.5


def _flat(r):
    ri = jax.lax.broadcasted_iota(jnp.int32, (r, 128), 0)
    ci = jax.lax.broadcasted_iota(jnp.int32, (r, 128), 1)
    return ri * 128 + ci


_NA = 6


def _exchange(stk, lj, dirm):
    rows = stk.shape[0]
    r = rows // _NA
    d = 1 << lj
    if d < 128:
        ci = jax.lax.broadcasted_iota(jnp.int32, (rows, 128), 1)
        bit_f = (ci & d) != 0
        p = jnp.where(bit_f, pltpu.roll(stk, d, 1), pltpu.roll(stk, 128 - d, 1))
    else:
        m = d >> 7
        ri = jax.lax.broadcasted_iota(jnp.int32, (rows, 128), 0)
        bit_f = (ri & m) != 0
        p = jnp.where(bit_f, pltpu.roll(stk, m, 0), pltpu.roll(stk, rows - m, 0))
    s, i = stk[0:r, :], stk[r:2 * r, :]
    ps, pi = p[0:r, :], p[r:2 * r, :]
    pref = (s > ps) | ((s == ps) & (i < pi))
    km = ~(pref ^ dirm ^ bit_f[0:r, :])
    km_f = jnp.concatenate([km] * _NA, axis=0)
    return jnp.where(km_f, stk, p)


def _topk_sort(stk):
    fi = _flat(_R0)
    for lk in range(1, 11):
        dirm = (fi & (1 << lk)) == 0
        for lj in range(lk - 1, -1, -1):
            stk = _exchange(stk, lj, dirm)

    size = _NP
    while size > _KP:
        rows = stk.shape[0]
        stk = _exchange(stk, 10, True)
        stk = stk.reshape(rows // 16, 2, 8, 128)[:, 0].reshape(rows // 2, 128)
        size //= 2
        final = size == _KP
        dirm = True if final else (_flat(rows // (2 * _NA)) & _KP) == 0
        for lj in range(9, -1, -1):
            stk = _exchange(stk, lj, dirm)
    return stk


def _rowform(a):
    return jnp.concatenate([a[i:i + 1, :] for i in range(8)], axis=1)


def _colform(a):
    b = jnp.broadcast_to(a[:, None, :], (8, 128, 128)).reshape(_KP, 128)
    ri = jax.lax.broadcasted_iota(jnp.int32, (_KP, 128), 0)
    ci = jax.lax.broadcasted_iota(jnp.int32, (_KP, 128), 1)
    sel = ci == (ri & 127)
    return jnp.sum(jnp.where(sel, b, 0.0), axis=1, keepdims=True)


def _body(score_ref, x1_ref, y1_ref, x2_ref, y2_ref, init_ref, out_ref):
    idx0 = _flat(_R0).astype(jnp.float32)
    stk = jnp.concatenate(
        [score_ref[...], idx0,
         x1_ref[...], y1_ref[...], x2_ref[...], y2_ref[...]], axis=0)
    top = _topk_sort(stk)
    s = top[0:8, :]
    x1 = top[16:24, :]
    y1 = top[24:32, :]
    x2 = top[32:40, :]
    y2 = top[40:48, :]

    sr = _rowform(s)
    x1r = _rowform(x1)
    y1r = _rowform(y1)
    x2r = _rowform(x2)
    y2r = _rowform(y2)
    x1c = _colform(x1)
    y1c = _colform(y1)
    x2c = _colform(x2)
    y2c = _colform(y2)

    area_c = (x2c - x1c) * (y2c - y1c)
    area_r = (x2r - x1r) * (y2r - y1r)
    xx1 = jnp.maximum(x1c, x1r)
    yy1 = jnp.maximum(y1c, y1r)
    xx2 = jnp.minimum(x2c, x2r)
    yy2 = jnp.minimum(y2c, y2r)
    iw = jnp.clip(xx2 - xx1, 0.0, None)
    ih = jnp.clip(yy2 - yy1, 0.0, None)
    inter = iw * ih
    union = area_c + area_r - inter
    iou = inter / (union + 1e-8)

    ii = jax.lax.broadcasted_iota(jnp.int32, (_KP, _KP), 0)
    jj = jax.lax.broadcasted_iota(jnp.int32, (_KP, _KP), 1)
    sup = jnp.where((iou > _T) & (ii < jj), 1.0, 0.0)

    init = init_ref[...]

    def cond(c):
        return c[1]

    def body(c):
        keep, _ = c
        hits = jax.lax.dot_general(
            keep, sup, (((1,), (0,)), ((), ())),
            preferred_element_type=jnp.float32)
        new = jnp.where(hits == 0.0, init, 0.0)
        return new, jnp.any(new != keep)

    keep, _ = jax.lax.while_loop(cond, body, (init, jnp.bool_(True)))

    out_ref[0:1, :] = x1r * keep
    out_ref[1:2, :] = y1r * keep
    out_ref[2:3, :] = x2r * keep
    out_ref[3:4, :] = y2r * keep
    out_ref[4:5, :] = sr * keep
    out_ref[5:8, :] = jnp.zeros((3, _KP), jnp.float32)


def kernel(boxes, scores, k):
    pad = _NP - _N
    sp = jnp.pad(scores, (0, pad), constant_values=-1.0).reshape(_R0, 128)
    x1 = jnp.pad(boxes[:, 0], (0, pad)).reshape(_R0, 128)
    y1 = jnp.pad(boxes[:, 1], (0, pad)).reshape(_R0, 128)
    x2 = jnp.pad(boxes[:, 2], (0, pad)).reshape(_R0, 128)
    y2 = jnp.pad(boxes[:, 3], (0, pad)).reshape(_R0, 128)
    init = (jnp.arange(_KP) < k).astype(jnp.float32).reshape(1, _KP)

    out = pl.pallas_call(
        _body,
        out_shape=jax.ShapeDtypeStruct((8, _KP), jnp.float32),
    )(sp, x1, y1, x2, y2, init)

    return out.T[:_K, :5]
